# baseline (speedup 1.0000x reference)
import functools

import jax
import jax.numpy as jnp
from jax import lax
from jax.experimental import pallas as pl
from jax.experimental.pallas import tpu as pltpu
from jax.experimental.pallas import tpu_sc as plsc

B = 16384
D = 16
NC = 2
NS = 16
NW = NC * NS
BPW = B // NW
CB = 128
CHUNKS = BPW // CB


@functools.partial(
    pl.kernel,
    mesh=plsc.VectorSubcoreMesh(core_axis_name="c", subcore_axis_name="s"),
    out_type=jax.ShapeDtypeStruct((B,), jnp.float32),
    compiler_params=pltpu.CompilerParams(
        needs_layout_passes=False, use_tc_tiling_on_sc=False),
    scratch_types=[
        pltpu.VMEM((CHUNKS, CB), jnp.int32),
        pltpu.VMEM((CHUNKS, CB), jnp.int32),
        pltpu.VMEM((BPW, D), jnp.float32),
        pltpu.VMEM((BPW, D), jnp.float32),
        pltpu.VMEM((16 * D,), jnp.float32),
        pltpu.VMEM((BPW,), jnp.float32),
        pltpu.SemaphoreType.DMA,
    ],
)
def _bpr_sc(lidx_hbm, iidx_hbm, lemb_hbm, iemb_hbm, out_hbm,
            lidx_v, iidx_v, lrows_v, irows_v, prod_v, out_v, sem):
    wid = lax.axis_index("s") * NC + lax.axis_index("c")
    base = wid * BPW

    pltpu.sync_copy(lidx_hbm.at[wid], lidx_v)
    pltpu.sync_copy(iidx_hbm.at[wid], iidx_v)

    copies = []
    for j in range(CHUNKS):
        copies.append(
            pltpu.async_copy(lemb_hbm.at[lidx_v.at[j]],
                             lrows_v.at[pl.ds(j * CB, CB)], sem))
        copies.append(
            pltpu.async_copy(iemb_hbm.at[iidx_v.at[j]],
                             irows_v.at[pl.ds(j * CB, CB)], sem))
    for c in copies:
        c.wait()

    lane = lax.iota(jnp.int32, 16)
    cols = [lane * D + d for d in range(D)]

    def block(t, carry):
        r0 = t * 16
        for r in range(16):
            prod_v[pl.ds(r * D, D)] = lrows_v[r0 + r] * irows_v[r0 + r]
        acc = None
        for d in range(D):
            pv = plsc.load_gather(prod_v, [cols[d]])
            acc = pv if acc is None else acc + pv
        out_v[pl.ds(r0, 16)] = acc
        return carry

    lax.fori_loop(0, BPW // 16, block, 0)

    pltpu.sync_copy(out_v, out_hbm.at[pl.ds(base, BPW)])


def kernel(user_pos_indices, user_neg_indices, list_indices, item_indices,
           list_emb, item_emb):
    lidx = list_indices.astype(jnp.int32).reshape(NW, CHUNKS, CB)
    iidx = item_indices.astype(jnp.int32).reshape(NW, CHUNKS, CB)
    return _bpr_sc(lidx, iidx, list_emb, item_emb)
